# baseline (speedup 1.0000x reference)
import functools

import jax
import jax.numpy as jnp
from jax import lax
from jax.experimental import pallas as pl
from jax.experimental.pallas import tpu as pltpu
from jax.experimental.pallas import tpu_sc as plsc

N = 10000
E = 320000
D = 128
NCLS = 16
NET = 8
NPAD = 10240
NCORES = 2
NSUB = 16
CHUNK = 128
EP = 327680
NCHP = EP // CHUNK
CPCC = NCHP // NCORES
CONV_CPT = CPCC // NSUB
PREP_CPT = NCHP // NSUB
ROWS_PER_TILE = NPAD // NSUB
ACC = NPAD + 64
ZROWS = 64
IRING = 4
LAG = 3

RB = 512
GRID = NPAD // RB


def _mesh():
    return plsc.VectorSubcoreMesh(core_axis_name="c", subcore_axis_name="s")


def _sc_prep(src2, dst2, ef2, ones_in, zeros_in):
    @functools.partial(
        pl.kernel,
        out_type=(
            jax.ShapeDtypeStruct((NCORES, NPAD, D), jnp.float32),
            jax.ShapeDtypeStruct((EP,), jnp.int32),
        ),
        mesh=_mesh(),
        scratch_types=[
            pltpu.VMEM((IRING, CHUNK), jnp.int32),
            pltpu.VMEM((CHUNK,), jnp.int32),
            pltpu.VMEM((CHUNK,), jnp.int32),
            pltpu.VMEM((CHUNK, D), jnp.float32),
            pltpu.VMEM((ZROWS, D), jnp.float32),
            pltpu.VMEM_SHARED((ACC, D), jnp.float32),
            pltpu.SemaphoreType.DMA,
        ],
    )
    def prep(src_h, dst_h, ef_h, ones_h, zeros_h, deg_h, gidx_h,
             idx_v, ef_v, gx_v, ones_v, zz_v, deg_sh, sems):
        c = lax.axis_index("c")
        s = lax.axis_index("s")
        cbase = s * PREP_CPT

        pltpu.sync_copy(ones_h, ones_v)
        pltpu.sync_copy(zeros_h, zz_v)

        def zslice(k, _):
            pltpu.sync_copy(zz_v, deg_sh.at[pl.ds(s * ROWS_PER_TILE + k * ZROWS, ZROWS)])
            return 0
        lax.fori_loop(0, ROWS_PER_TILE // ZROWS, zslice, 0)

        @pl.when(s == 0)
        def _():
            pltpu.sync_copy(zz_v, deg_sh.at[pl.ds(NPAD, ZROWS)])
        plsc.subcore_barrier()

        def drain_one():
            pltpu.make_async_copy(ones_h, ones_v, sems).wait()

        def body(it, _):
            ebase = (cbase + it) * CHUNK
            slot = lax.rem(it, IRING)

            @pl.when(it >= LAG)
            def _():
                drain_one()

            @pl.when(c == 0)
            def _():
                pltpu.sync_copy(src_h.at[pl.ds(ebase, CHUNK)], idx_v.at[slot])
                pltpu.sync_copy(ef_h.at[pl.ds(ebase, CHUNK)], ef_v)

                def g(j, _):
                    sl = pl.ds(j * 16, 16)
                    gx_v[sl] = ef_v[sl] * NPAD + idx_v[slot, sl]
                    return 0
                lax.fori_loop(0, CHUNK // 16, g, 0)
                pltpu.sync_copy(gx_v, gidx_h.at[pl.ds(ebase, CHUNK)])

            @pl.when(c == 1)
            def _():
                pltpu.sync_copy(dst_h.at[pl.ds(ebase, CHUNK)], idx_v.at[slot])

            pltpu.async_copy(ones_v, deg_sh.at[idx_v.at[slot]], sems, add=True)
            return 0
        lax.fori_loop(0, PREP_CPT, body, 0)
        for _ in range(LAG):
            drain_one()

        plsc.subcore_barrier()
        rbase = s * ROWS_PER_TILE
        pltpu.sync_copy(deg_sh.at[pl.ds(rbase, ROWS_PER_TILE)],
                        deg_h.at[c].at[pl.ds(rbase, ROWS_PER_TILE)])

    return prep(src2, dst2, ef2, ones_in, zeros_in)


def _sc_conv(tbl, gidx2, dst2):
    @functools.partial(
        pl.kernel,
        out_type=jax.ShapeDtypeStruct((NCORES, NPAD, D), jnp.float32),
        mesh=_mesh(),
        scratch_types=[
            pltpu.VMEM((2, CHUNK), jnp.int32),
            pltpu.VMEM((2, CHUNK), jnp.int32),
            pltpu.VMEM((2, CHUNK, D), jnp.float32),
            pltpu.VMEM((ZROWS, D), jnp.float32),
            pltpu.VMEM_SHARED((ACC, D), jnp.float32),
            pltpu.SemaphoreType.DMA,
            pltpu.SemaphoreType.DMA,
        ],
    )
    def conv(tbl_h, gidx_h, dst_h, out_h, gx_v, dst_v, rows_v, zz_v, agg_sh,
             semg, sems):
        c = lax.axis_index("c")
        s = lax.axis_index("s")
        cbase = c * CPCC + s * CONV_CPT

        zero16 = jnp.zeros((16,), jnp.float32)

        def fill_zeros(k, _):
            zz_v[k // (D // 16), pl.ds((k % (D // 16)) * 16, 16)] = zero16
            return 0
        lax.fori_loop(0, ZROWS * (D // 16), fill_zeros, 0)

        def zslice(k, _):
            pltpu.sync_copy(zz_v, agg_sh.at[pl.ds(s * ROWS_PER_TILE + k * ZROWS, ZROWS)])
            return 0
        lax.fori_loop(0, ROWS_PER_TILE // ZROWS, zslice, 0)

        @pl.when(s == 0)
        def _():
            pltpu.sync_copy(zz_v, agg_sh.at[pl.ds(NPAD, ZROWS)])
        plsc.subcore_barrier()

        def load_idx(j, slot):
            ebase = (cbase + j) * CHUNK
            pltpu.sync_copy(gidx_h.at[pl.ds(ebase, CHUNK)], gx_v.at[slot])
            pltpu.sync_copy(dst_h.at[pl.ds(ebase, CHUNK)], dst_v.at[slot])

        def drain_gather():
            pltpu.make_async_copy(tbl_h.at[gx_v.at[0]], rows_v.at[0], semg).wait()

        def drain_scatter():
            pltpu.make_async_copy(rows_v.at[0], agg_sh.at[pl.ds(0, CHUNK)], sems).wait()

        load_idx(0, 0)
        pltpu.async_copy(tbl_h.at[gx_v.at[0]], rows_v.at[0], semg)

        def body(it, _):
            b = lax.rem(it, 2)
            nb = lax.rem(it + 1, 2)

            @pl.when(it >= 1)
            def _():
                drain_scatter()

            @pl.when(it + 1 < CONV_CPT)
            def _():
                load_idx(it + 1, nb)
                pltpu.async_copy(tbl_h.at[gx_v.at[nb]], rows_v.at[nb], semg)

            drain_gather()
            pltpu.async_copy(rows_v.at[b], agg_sh.at[dst_v.at[b]], sems, add=True)
            return 0
        lax.fori_loop(0, CONV_CPT, body, 0)
        drain_scatter()

        plsc.subcore_barrier()
        rbase = s * ROWS_PER_TILE
        pltpu.sync_copy(agg_sh.at[pl.ds(rbase, ROWS_PER_TILE)],
                        out_h.at[c].at[pl.ds(rbase, ROWS_PER_TILE)])

    return conv(tbl, gidx2, dst2)


def _tc_h0(x0p, wT, b):
    def body(x_ref, w_ref, b_ref, out_ref):
        out_ref[...] = jnp.dot(x_ref[...], w_ref[...],
                               preferred_element_type=jnp.float32) + b_ref[...]

    return pl.pallas_call(
        body,
        grid=(GRID,),
        in_specs=[pl.BlockSpec((RB, D), lambda i: (i, 0)),
                  pl.BlockSpec((D, D), lambda i: (0, 0)),
                  pl.BlockSpec((1, D), lambda i: (0, 0))],
        out_specs=pl.BlockSpec((RB, D), lambda i: (i, 0)),
        out_shape=jax.ShapeDtypeStruct((NPAD, D), jnp.float32),
    )(x0p, wT, b.reshape(1, D))


def _tc_t0(h0, deg, et):
    def body(h_ref, dg_ref, et_ref, out_ref, ns_ref, nd_ref):
        ns = lax.rsqrt(jnp.where(dg_ref[0] > 0, dg_ref[0], 1.0))
        nd = lax.rsqrt(jnp.where(dg_ref[1] > 0, dg_ref[1], 1.0))
        ns_ref[...] = ns
        nd_ref[...] = nd
        hs = h_ref[...] * ns
        for t in range(NET):
            out_ref[t] = hs * et_ref[t]

    return pl.pallas_call(
        body,
        grid=(GRID,),
        in_specs=[pl.BlockSpec((RB, D), lambda i: (i, 0)),
                  pl.BlockSpec((NCORES, RB, D), lambda i: (0, i, 0)),
                  pl.BlockSpec(memory_space=pltpu.SMEM)],
        out_specs=[pl.BlockSpec((NET, RB, D), lambda i: (0, i, 0)),
                   pl.BlockSpec((RB, D), lambda i: (i, 0)),
                   pl.BlockSpec((RB, D), lambda i: (i, 0))],
        out_shape=[jax.ShapeDtypeStruct((NET, NPAD, D), jnp.float32),
                   jax.ShapeDtypeStruct((NPAD, D), jnp.float32),
                   jax.ShapeDtypeStruct((NPAD, D), jnp.float32)],
    )(h0, deg, et)


def _tc_table_l1(agg, nd, ns, et):
    def body(a_ref, nd_ref, ns_ref, et_ref, out_ref):
        hs = (a_ref[0] + a_ref[1]) * nd_ref[...] * ns_ref[...]
        for t in range(NET):
            out_ref[t] = hs * et_ref[t]

    return pl.pallas_call(
        body,
        grid=(GRID,),
        in_specs=[pl.BlockSpec((NCORES, RB, D), lambda i: (0, i, 0)),
                  pl.BlockSpec((RB, D), lambda i: (i, 0)),
                  pl.BlockSpec((RB, D), lambda i: (i, 0)),
                  pl.BlockSpec(memory_space=pltpu.SMEM)],
        out_specs=pl.BlockSpec((NET, RB, D), lambda i: (0, i, 0)),
        out_shape=jax.ShapeDtypeStruct((NET, NPAD, D), jnp.float32),
    )(agg, nd, ns, et)


def _tc_table_l2(agg, nd, w1, b1, ns, et):
    def body(a_ref, nd_ref, w_ref, b_ref, ns_ref, et_ref, out_ref):
        hin = (a_ref[0] + a_ref[1]) * nd_ref[...]
        h = jnp.dot(hin, w_ref[...], preferred_element_type=jnp.float32) + b_ref[...]
        hs = jnp.maximum(h, 0.0) * ns_ref[...]
        for t in range(NET):
            out_ref[t] = hs * et_ref[t]

    return pl.pallas_call(
        body,
        grid=(GRID,),
        in_specs=[pl.BlockSpec((NCORES, RB, D), lambda i: (0, i, 0)),
                  pl.BlockSpec((RB, D), lambda i: (i, 0)),
                  pl.BlockSpec((D, D), lambda i: (0, 0)),
                  pl.BlockSpec((1, D), lambda i: (0, 0)),
                  pl.BlockSpec((RB, D), lambda i: (i, 0)),
                  pl.BlockSpec(memory_space=pltpu.SMEM)],
        out_specs=pl.BlockSpec((NET, RB, D), lambda i: (0, i, 0)),
        out_shape=jax.ShapeDtypeStruct((NET, NPAD, D), jnp.float32),
    )(agg, nd, w1, b1.reshape(1, D), ns, et)


def _tc_final(agg, nd, w2p, b2p):
    def body(a_ref, nd_ref, w_ref, b_ref, out_ref):
        hin = (a_ref[0] + a_ref[1]) * nd_ref[...]
        out_ref[...] = jnp.dot(hin, w_ref[...],
                               preferred_element_type=jnp.float32) + b_ref[...]

    return pl.pallas_call(
        body,
        grid=(GRID,),
        in_specs=[pl.BlockSpec((NCORES, RB, D), lambda i: (0, i, 0)),
                  pl.BlockSpec((RB, D), lambda i: (i, 0)),
                  pl.BlockSpec((D, D), lambda i: (0, 0)),
                  pl.BlockSpec((1, D), lambda i: (0, 0))],
        out_specs=pl.BlockSpec((RB, D), lambda i: (i, 0)),
        out_shape=jax.ShapeDtypeStruct((NPAD, D), jnp.float32),
    )(agg, nd, w2p, b2p.reshape(1, D))


def kernel(x0, edge_index, e_feat, W_fc0, b_fc0, et0, et1, et2, W1, b1, W2, b2):
    src = edge_index[0]
    dst = edge_index[1]
    pad = EP - E
    padidx = jnp.full((pad,), NPAD, jnp.int32)
    src2 = jnp.concatenate([src, padidx])
    dst2 = jnp.concatenate([dst, padidx])
    ef2 = jnp.concatenate([e_feat, jnp.zeros((pad,), jnp.int32)])
    x0p = jnp.pad(x0, ((0, NPAD - N), (0, 0)))
    w2p = jnp.pad(W2, ((0, 0), (0, D - NCLS)))
    b2p = jnp.pad(b2, ((0, D - NCLS),))
    ones_in = jnp.ones((CHUNK, D), jnp.float32)
    zeros_in = jnp.zeros((ZROWS, D), jnp.float32)

    deg, gidx2 = _sc_prep(src2, dst2, ef2, ones_in, zeros_in)
    h0 = _tc_h0(x0p, W_fc0.T, b_fc0)
    tbl0, ns, nd = _tc_t0(h0, deg, et0)

    agg0 = _sc_conv(tbl0.reshape(NET * NPAD, D), gidx2, dst2)
    tbl1 = _tc_table_l1(agg0, nd, ns, et1).reshape(NET * NPAD, D)
    agg1 = _sc_conv(tbl1, gidx2, dst2)
    tbl2 = _tc_table_l2(agg1, nd, W1, b1, ns, et2).reshape(NET * NPAD, D)
    agg2 = _sc_conv(tbl2, gidx2, dst2)
    out = _tc_final(agg2, nd, w2p, b2p)
    return out[:N, :NCLS]


# speedup vs baseline: 2.5080x; 2.5080x over previous
import functools

import jax
import jax.numpy as jnp
from jax import lax
from jax.experimental import pallas as pl
from jax.experimental.pallas import tpu as pltpu
from jax.experimental.pallas import tpu_sc as plsc

N = 10000
E = 320000
D = 128
NCLS = 16
NET = 8
NPAD = 10240
NCORES = 2
NSUB = 16
CHUNK = 128
EP = 327680
NCHP = EP // CHUNK
CPCC = NCHP // NCORES
CONV_CPT = CPCC // NSUB
PREP_CPT = NCHP // NSUB
ROWS_PER_TILE = NPAD // NSUB
ACC = NPAD + 64
ZROWS = 64
IRING = 4
LAG = 3

RB = 512
GRID = NPAD // RB


def _mesh():
    return plsc.VectorSubcoreMesh(core_axis_name="c", subcore_axis_name="s")


def _sc_prep(src2, dst2, ef2, ones_in, zeros_in):
    @functools.partial(
        pl.kernel,
        out_type=(
            jax.ShapeDtypeStruct((NCORES, NPAD, D), jnp.float32),
            jax.ShapeDtypeStruct((EP,), jnp.int32),
        ),
        mesh=_mesh(),
        scratch_types=[
            pltpu.VMEM((IRING, CHUNK), jnp.int32),
            pltpu.VMEM((CHUNK,), jnp.int32),
            pltpu.VMEM((CHUNK,), jnp.int32),
            pltpu.VMEM((CHUNK, D), jnp.float32),
            pltpu.VMEM((ZROWS, D), jnp.float32),
            pltpu.VMEM_SHARED((ACC, D), jnp.float32),
            pltpu.SemaphoreType.DMA,
        ],
    )
    def prep(src_h, dst_h, ef_h, ones_h, zeros_h, deg_h, gidx_h,
             idx_v, ef_v, gx_v, ones_v, zz_v, deg_sh, sems):
        c = lax.axis_index("c")
        s = lax.axis_index("s")
        cbase = s * PREP_CPT

        pltpu.sync_copy(ones_h, ones_v)
        pltpu.sync_copy(zeros_h, zz_v)

        def zslice(k, _):
            pltpu.sync_copy(zz_v, deg_sh.at[pl.ds(s * ROWS_PER_TILE + k * ZROWS, ZROWS)])
            return 0
        lax.fori_loop(0, ROWS_PER_TILE // ZROWS, zslice, 0)

        @pl.when(s == 0)
        def _():
            pltpu.sync_copy(zz_v, deg_sh.at[pl.ds(NPAD, ZROWS)])
        plsc.subcore_barrier()

        def drain_one():
            pltpu.make_async_copy(ones_h, ones_v, sems).wait()

        def body(it, _):
            ebase = (cbase + it) * CHUNK
            slot = lax.rem(it, IRING)

            @pl.when(it >= LAG)
            def _():
                drain_one()

            @pl.when(c == 0)
            def _():
                pltpu.sync_copy(src_h.at[pl.ds(ebase, CHUNK)], idx_v.at[slot])
                pltpu.sync_copy(ef_h.at[pl.ds(ebase, CHUNK)], ef_v)

                def g(j, _):
                    sl = pl.ds(j * 16, 16)
                    gx_v[sl] = ef_v[sl] * NPAD + idx_v[slot, sl]
                    return 0
                lax.fori_loop(0, CHUNK // 16, g, 0)
                pltpu.sync_copy(gx_v, gidx_h.at[pl.ds(ebase, CHUNK)])

            @pl.when(c == 1)
            def _():
                pltpu.sync_copy(dst_h.at[pl.ds(ebase, CHUNK)], idx_v.at[slot])

            pltpu.async_copy(ones_v, deg_sh.at[idx_v.at[slot]], sems, add=True)
            return 0
        lax.fori_loop(0, PREP_CPT, body, 0)
        for _ in range(LAG):
            drain_one()

        plsc.subcore_barrier()
        rbase = s * ROWS_PER_TILE
        pltpu.sync_copy(deg_sh.at[pl.ds(rbase, ROWS_PER_TILE)],
                        deg_h.at[c].at[pl.ds(rbase, ROWS_PER_TILE)])

    return prep(src2, dst2, ef2, ones_in, zeros_in)


def _sc_conv(tbl, gidx2, dst2):
    @functools.partial(
        pl.kernel,
        out_type=jax.ShapeDtypeStruct((NCORES, NPAD, D), jnp.float32),
        mesh=_mesh(),
        scratch_types=[
            pltpu.VMEM((2, CHUNK), jnp.int32),
            pltpu.VMEM((2, CHUNK), jnp.int32),
            pltpu.VMEM((2, CHUNK, D), jnp.float32),
            pltpu.VMEM((ZROWS, D), jnp.float32),
            pltpu.VMEM_SHARED((ACC, D), jnp.float32),
            pltpu.SemaphoreType.DMA,
            pltpu.SemaphoreType.DMA,
        ],
    )
    def conv(tbl_h, gidx_h, dst_h, out_h, gx_v, dst_v, rows_v, zz_v, agg_sh,
             semg, sems):
        c = lax.axis_index("c")
        s = lax.axis_index("s")
        cbase = c * CPCC + s * CONV_CPT

        zero16 = jnp.zeros((16,), jnp.float32)

        def fill_zeros(k, _):
            zz_v[k // (D // 16), pl.ds((k % (D // 16)) * 16, 16)] = zero16
            return 0
        lax.fori_loop(0, ZROWS * (D // 16), fill_zeros, 0)

        def zslice(k, _):
            pltpu.sync_copy(zz_v, agg_sh.at[pl.ds(s * ROWS_PER_TILE + k * ZROWS, ZROWS)])
            return 0
        lax.fori_loop(0, ROWS_PER_TILE // ZROWS, zslice, 0)

        @pl.when(s == 0)
        def _():
            pltpu.sync_copy(zz_v, agg_sh.at[pl.ds(NPAD, ZROWS)])
        plsc.subcore_barrier()

        def load_idx(j, slot):
            ebase = (cbase + j) * CHUNK
            pltpu.sync_copy(gidx_h.at[pl.ds(ebase, CHUNK)], gx_v.at[slot])
            pltpu.sync_copy(dst_h.at[pl.ds(ebase, CHUNK)], dst_v.at[slot])

        def drain_gather():
            pltpu.make_async_copy(tbl_h.at[gx_v.at[0]], rows_v.at[0], semg).wait()

        def drain_scatter():
            pltpu.make_async_copy(rows_v.at[0], agg_sh.at[pl.ds(0, CHUNK)], sems).wait()

        load_idx(0, 0)
        pltpu.async_copy(tbl_h.at[gx_v.at[0]], rows_v.at[0], semg)

        def body(it, _):
            b = lax.rem(it, 2)
            nb = lax.rem(it + 1, 2)

            @pl.when(it >= 1)
            def _():
                drain_scatter()

            @pl.when(it + 1 < CONV_CPT)
            def _():
                load_idx(it + 1, nb)
                pltpu.async_copy(tbl_h.at[gx_v.at[nb]], rows_v.at[nb], semg)

            drain_gather()
            pltpu.async_copy(rows_v.at[b], agg_sh.at[dst_v.at[b]], sems, add=True)
            return 0
        lax.fori_loop(0, CONV_CPT, body, 0)
        drain_scatter()

        plsc.subcore_barrier()
        rbase = s * ROWS_PER_TILE
        pltpu.sync_copy(agg_sh.at[pl.ds(rbase, ROWS_PER_TILE)],
                        out_h.at[c].at[pl.ds(rbase, ROWS_PER_TILE)])

    return conv(tbl, gidx2, dst2)


def _tc_h0(x0p, wT, b):
    def body(x_ref, w_ref, b_ref, out_ref):
        out_ref[...] = jnp.dot(x_ref[...], w_ref[...],
                               preferred_element_type=jnp.float32) + b_ref[...]

    return pl.pallas_call(
        body,
        grid=(GRID,),
        in_specs=[pl.BlockSpec((RB, D), lambda i: (i, 0)),
                  pl.BlockSpec((D, D), lambda i: (0, 0)),
                  pl.BlockSpec((1, D), lambda i: (0, 0))],
        out_specs=pl.BlockSpec((RB, D), lambda i: (i, 0)),
        out_shape=jax.ShapeDtypeStruct((NPAD, D), jnp.float32),
    )(x0p, wT, b.reshape(1, D))


def _tc_t0(h0, deg, et):
    def body(h_ref, dg_ref, et_ref, out_ref, ns_ref, nd_ref):
        ns = lax.rsqrt(jnp.where(dg_ref[0] > 0, dg_ref[0], 1.0))
        nd = lax.rsqrt(jnp.where(dg_ref[1] > 0, dg_ref[1], 1.0))
        ns_ref[...] = ns
        nd_ref[...] = nd
        hs = h_ref[...] * ns
        for t in range(NET):
            out_ref[t] = hs * et_ref[t]

    return pl.pallas_call(
        body,
        grid=(GRID,),
        in_specs=[pl.BlockSpec((RB, D), lambda i: (i, 0)),
                  pl.BlockSpec((NCORES, RB, D), lambda i: (0, i, 0)),
                  pl.BlockSpec(memory_space=pltpu.SMEM)],
        out_specs=[pl.BlockSpec((NET, RB, D), lambda i: (0, i, 0)),
                   pl.BlockSpec((RB, D), lambda i: (i, 0)),
                   pl.BlockSpec((RB, D), lambda i: (i, 0))],
        out_shape=[jax.ShapeDtypeStruct((NET, NPAD, D), jnp.float32),
                   jax.ShapeDtypeStruct((NPAD, D), jnp.float32),
                   jax.ShapeDtypeStruct((NPAD, D), jnp.float32)],
    )(h0, deg, et)


def _tc_table_l1(agg, nd, ns, et):
    def body(a_ref, nd_ref, ns_ref, et_ref, out_ref):
        hs = (a_ref[0] + a_ref[1]) * nd_ref[...] * ns_ref[...]
        for t in range(NET):
            out_ref[t] = hs * et_ref[t]

    return pl.pallas_call(
        body,
        grid=(GRID,),
        in_specs=[pl.BlockSpec((NCORES, RB, D), lambda i: (0, i, 0)),
                  pl.BlockSpec((RB, D), lambda i: (i, 0)),
                  pl.BlockSpec((RB, D), lambda i: (i, 0)),
                  pl.BlockSpec(memory_space=pltpu.SMEM)],
        out_specs=pl.BlockSpec((NET, RB, D), lambda i: (0, i, 0)),
        out_shape=jax.ShapeDtypeStruct((NET, NPAD, D), jnp.float32),
    )(agg, nd, ns, et)


def _tc_table_l2(agg, nd, w1, b1, ns, et):
    def body(a_ref, nd_ref, w_ref, b_ref, ns_ref, et_ref, out_ref):
        hin = (a_ref[0] + a_ref[1]) * nd_ref[...]
        h = jnp.dot(hin, w_ref[...], preferred_element_type=jnp.float32) + b_ref[...]
        hs = jnp.maximum(h, 0.0) * ns_ref[...]
        for t in range(NET):
            out_ref[t] = hs * et_ref[t]

    return pl.pallas_call(
        body,
        grid=(GRID,),
        in_specs=[pl.BlockSpec((NCORES, RB, D), lambda i: (0, i, 0)),
                  pl.BlockSpec((RB, D), lambda i: (i, 0)),
                  pl.BlockSpec((D, D), lambda i: (0, 0)),
                  pl.BlockSpec((1, D), lambda i: (0, 0)),
                  pl.BlockSpec((RB, D), lambda i: (i, 0)),
                  pl.BlockSpec(memory_space=pltpu.SMEM)],
        out_specs=pl.BlockSpec((NET, RB, D), lambda i: (0, i, 0)),
        out_shape=jax.ShapeDtypeStruct((NET, NPAD, D), jnp.float32),
    )(agg, nd, w1, b1.reshape(1, D), ns, et)


def _tc_final(agg, nd, w2p, b2p):
    def body(a_ref, nd_ref, w_ref, b_ref, out_ref):
        hin = (a_ref[0] + a_ref[1]) * nd_ref[...]
        out_ref[...] = jnp.dot(hin, w_ref[...],
                               preferred_element_type=jnp.float32) + b_ref[...]

    return pl.pallas_call(
        body,
        grid=(GRID,),
        in_specs=[pl.BlockSpec((NCORES, RB, D), lambda i: (0, i, 0)),
                  pl.BlockSpec((RB, D), lambda i: (i, 0)),
                  pl.BlockSpec((D, D), lambda i: (0, 0)),
                  pl.BlockSpec((1, D), lambda i: (0, 0))],
        out_specs=pl.BlockSpec((RB, D), lambda i: (i, 0)),
        out_shape=jax.ShapeDtypeStruct((NPAD, D), jnp.float32),
    )(agg, nd, w2p, b2p.reshape(1, D))


def kernel(x0, edge_index, e_feat, W_fc0, b_fc0, et0, et1, et2, W1, b1, W2, b2):
    src = edge_index[0]
    dst = edge_index[1]
    pad = EP - E
    padidx = NPAD + (jnp.arange(pad, dtype=jnp.int32) % 64)
    src2 = jnp.concatenate([src, padidx])
    dst2 = jnp.concatenate([dst, padidx])
    ef2 = jnp.concatenate([e_feat, jnp.zeros((pad,), jnp.int32)])
    x0p = jnp.pad(x0, ((0, NPAD - N), (0, 0)))
    w2p = jnp.pad(W2, ((0, 0), (0, D - NCLS)))
    b2p = jnp.pad(b2, ((0, D - NCLS),))
    ones_in = jnp.ones((CHUNK, D), jnp.float32)
    zeros_in = jnp.zeros((ZROWS, D), jnp.float32)

    deg, gidx2 = _sc_prep(src2, dst2, ef2, ones_in, zeros_in)
    h0 = _tc_h0(x0p, W_fc0.T, b_fc0)
    tbl0, ns, nd = _tc_t0(h0, deg, et0)

    agg0 = _sc_conv(tbl0.reshape(NET * NPAD, D), gidx2, dst2)
    tbl1 = _tc_table_l1(agg0, nd, ns, et1).reshape(NET * NPAD, D)
    agg1 = _sc_conv(tbl1, gidx2, dst2)
    tbl2 = _tc_table_l2(agg1, nd, W1, b1, ns, et2).reshape(NET * NPAD, D)
    agg2 = _sc_conv(tbl2, gidx2, dst2)
    out = _tc_final(agg2, nd, w2p, b2p)
    return out[:N, :NCLS]


# speedup vs baseline: 3.3153x; 1.3218x over previous
import functools

import jax
import jax.numpy as jnp
from jax import lax
from jax.experimental import pallas as pl
from jax.experimental.pallas import tpu as pltpu
from jax.experimental.pallas import tpu_sc as plsc

N = 10000
E = 320000
D = 128
NCLS = 16
NET = 8
NPAD = 10240
NCORES = 2
NSUB = 16
CHUNK = 128
EP = 327680
NCHP = EP // CHUNK
CPCC = NCHP // NCORES
CONV_CPT = CPCC // NSUB
PREP_CPT = NCHP // NSUB
ROWS_PER_TILE = NPAD // NSUB
ACC = NPAD + 64
ZROWS = 64
IRING = 8
LAG = 3

RB = 512
GRID = NPAD // RB


def _mesh():
    return plsc.VectorSubcoreMesh(core_axis_name="c", subcore_axis_name="s")


def _sc_prep(src2, dst2, ef2, ones_in, zeros_in):
    @functools.partial(
        pl.kernel,
        out_type=(
            jax.ShapeDtypeStruct((NCORES, NPAD, D), jnp.float32),
            jax.ShapeDtypeStruct((EP,), jnp.int32),
        ),
        mesh=_mesh(),
        scratch_types=[
            pltpu.VMEM((IRING, CHUNK), jnp.int32),
            pltpu.VMEM((IRING, CHUNK), jnp.int32),
            pltpu.VMEM((2, CHUNK), jnp.int32),
            pltpu.VMEM((CHUNK, D), jnp.float32),
            pltpu.VMEM((ZROWS, D), jnp.float32),
            pltpu.VMEM_SHARED((ACC, D), jnp.float32),
            pltpu.SemaphoreType.DMA,
            pltpu.SemaphoreType.DMA,
            pltpu.SemaphoreType.DMA,
            pltpu.SemaphoreType.DMA,
        ],
    )
    def prep(src_h, dst_h, ef_h, ones_h, zeros_h, deg_h, gidx_h,
             idx_v, ef_v, gx_v, ones_v, zz_v, deg_sh, sems, semi, seme, semw):
        c = lax.axis_index("c")
        s = lax.axis_index("s")
        cbase = s * PREP_CPT

        pltpu.sync_copy(ones_h, ones_v)
        pltpu.sync_copy(zeros_h, zz_v)

        def zslice(k, _):
            pltpu.sync_copy(zz_v, deg_sh.at[pl.ds(s * ROWS_PER_TILE + k * ZROWS, ZROWS)])
            return 0
        lax.fori_loop(0, ROWS_PER_TILE // ZROWS, zslice, 0)

        @pl.when(s == 0)
        def _():
            pltpu.sync_copy(zz_v, deg_sh.at[pl.ds(NPAD, ZROWS)])
        plsc.subcore_barrier()

        def fire_idx(j):
            ebase = (cbase + j) * CHUNK
            slot = lax.rem(j, IRING)

            @pl.when(c == 0)
            def _():
                pltpu.async_copy(src_h.at[pl.ds(ebase, CHUNK)], idx_v.at[slot], semi)
                pltpu.async_copy(ef_h.at[pl.ds(ebase, CHUNK)], ef_v.at[slot], seme)

            @pl.when(c == 1)
            def _():
                pltpu.async_copy(dst_h.at[pl.ds(ebase, CHUNK)], idx_v.at[slot], semi)

        def drain_idx():
            pltpu.make_async_copy(src_h.at[pl.ds(0, CHUNK)], idx_v.at[0], semi).wait()

            @pl.when(c == 0)
            def _():
                pltpu.make_async_copy(ef_h.at[pl.ds(0, CHUNK)], ef_v.at[0], seme).wait()

        def drain_scatter():
            pltpu.make_async_copy(ones_h, ones_v, sems).wait()

        def drain_gidx():
            pltpu.make_async_copy(gx_v.at[0], gidx_h.at[pl.ds(0, CHUNK)], semw).wait()

        for j in range(LAG):
            fire_idx(j)

        def body(it, _):
            slot = lax.rem(it, IRING)

            @pl.when(it >= LAG)
            def _():
                drain_scatter()
            drain_idx()
            pltpu.async_copy(ones_v, deg_sh.at[idx_v.at[slot]], sems, add=True)

            @pl.when(it + LAG < PREP_CPT)
            def _():
                fire_idx(it + LAG)

            @pl.when(c == 0)
            def _():
                gs = lax.rem(it, 2)

                @pl.when(it >= 2)
                def _():
                    drain_gidx()

                def g(j, _):
                    sl = pl.ds(j * 16, 16)
                    gx_v[gs, sl] = ef_v[slot, sl] * NPAD + idx_v[slot, sl]
                    return 0
                lax.fori_loop(0, CHUNK // 16, g, 0)
                pltpu.async_copy(gx_v.at[gs], gidx_h.at[pl.ds((cbase + it) * CHUNK, CHUNK)],
                                 semw)
            return 0
        lax.fori_loop(0, PREP_CPT, body, 0)
        for _ in range(LAG):
            drain_scatter()

        @pl.when(c == 0)
        def _():
            drain_gidx()
            drain_gidx()

        plsc.subcore_barrier()
        rbase = s * ROWS_PER_TILE
        pltpu.sync_copy(deg_sh.at[pl.ds(rbase, ROWS_PER_TILE)],
                        deg_h.at[c].at[pl.ds(rbase, ROWS_PER_TILE)])

    return prep(src2, dst2, ef2, ones_in, zeros_in)


def _sc_conv(tbl, gidx2, dst2):
    @functools.partial(
        pl.kernel,
        out_type=jax.ShapeDtypeStruct((NCORES, NPAD, D), jnp.float32),
        mesh=_mesh(),
        scratch_types=[
            pltpu.VMEM((4, CHUNK), jnp.int32),
            pltpu.VMEM((4, CHUNK), jnp.int32),
            pltpu.VMEM((2, CHUNK, D), jnp.float32),
            pltpu.VMEM((ZROWS, D), jnp.float32),
            pltpu.VMEM_SHARED((ACC, D), jnp.float32),
            pltpu.SemaphoreType.DMA,
            pltpu.SemaphoreType.DMA,
            pltpu.SemaphoreType.DMA,
        ],
    )
    def conv(tbl_h, gidx_h, dst_h, out_h, gx_v, dst_v, rows_v, zz_v, agg_sh,
             semg, sems, semi):
        c = lax.axis_index("c")
        s = lax.axis_index("s")
        cbase = c * CPCC + s * CONV_CPT

        zero16 = jnp.zeros((16,), jnp.float32)

        def fill_zeros(k, _):
            zz_v[k // (D // 16), pl.ds((k % (D // 16)) * 16, 16)] = zero16
            return 0
        lax.fori_loop(0, ZROWS * (D // 16), fill_zeros, 0)

        def zslice(k, _):
            pltpu.sync_copy(zz_v, agg_sh.at[pl.ds(s * ROWS_PER_TILE + k * ZROWS, ZROWS)])
            return 0
        lax.fori_loop(0, ROWS_PER_TILE // ZROWS, zslice, 0)

        @pl.when(s == 0)
        def _():
            pltpu.sync_copy(zz_v, agg_sh.at[pl.ds(NPAD, ZROWS)])
        plsc.subcore_barrier()

        def fire_idx(j):
            ebase = (cbase + j) * CHUNK
            slot = lax.rem(j, 4)
            pltpu.async_copy(gidx_h.at[pl.ds(ebase, CHUNK)], gx_v.at[slot], semi)
            pltpu.async_copy(dst_h.at[pl.ds(ebase, CHUNK)], dst_v.at[slot], semi)

        def drain_idx():
            pltpu.make_async_copy(gidx_h.at[pl.ds(0, CHUNK)], gx_v.at[0], semi).wait()
            pltpu.make_async_copy(dst_h.at[pl.ds(0, CHUNK)], dst_v.at[0], semi).wait()

        def drain_gather():
            pltpu.make_async_copy(tbl_h.at[gx_v.at[0]], rows_v.at[0], semg).wait()

        def drain_scatter():
            pltpu.make_async_copy(rows_v.at[0], agg_sh.at[pl.ds(0, CHUNK)], sems).wait()

        for j in range(3):
            fire_idx(j)
        drain_idx()
        pltpu.async_copy(tbl_h.at[gx_v.at[0]], rows_v.at[0], semg)

        def body(it, _):
            b = lax.rem(it, 2)
            nb = lax.rem(it + 1, 2)

            @pl.when(it >= 1)
            def _():
                drain_scatter()

            @pl.when(it + 1 < CONV_CPT)
            def _():
                drain_idx()
                pltpu.async_copy(tbl_h.at[gx_v.at[lax.rem(it + 1, 4)]],
                                 rows_v.at[nb], semg)

            @pl.when(it + 3 < CONV_CPT)
            def _():
                fire_idx(it + 3)

            drain_gather()
            pltpu.async_copy(rows_v.at[b], agg_sh.at[dst_v.at[lax.rem(it, 4)]],
                             sems, add=True)
            return 0
        lax.fori_loop(0, CONV_CPT, body, 0)
        drain_scatter()

        plsc.subcore_barrier()
        rbase = s * ROWS_PER_TILE
        pltpu.sync_copy(agg_sh.at[pl.ds(rbase, ROWS_PER_TILE)],
                        out_h.at[c].at[pl.ds(rbase, ROWS_PER_TILE)])

    return conv(tbl, gidx2, dst2)


def _tc_h0(x0p, wT, b):
    def body(x_ref, w_ref, b_ref, out_ref):
        out_ref[...] = jnp.dot(x_ref[...], w_ref[...],
                               preferred_element_type=jnp.float32) + b_ref[...]

    return pl.pallas_call(
        body,
        grid=(GRID,),
        in_specs=[pl.BlockSpec((RB, D), lambda i: (i, 0)),
                  pl.BlockSpec((D, D), lambda i: (0, 0)),
                  pl.BlockSpec((1, D), lambda i: (0, 0))],
        out_specs=pl.BlockSpec((RB, D), lambda i: (i, 0)),
        out_shape=jax.ShapeDtypeStruct((NPAD, D), jnp.float32),
    )(x0p, wT, b.reshape(1, D))


def _tc_t0(h0, deg, et):
    def body(h_ref, dg_ref, et_ref, out_ref, ns_ref, nd_ref):
        ns = lax.rsqrt(jnp.where(dg_ref[0] > 0, dg_ref[0], 1.0))
        nd = lax.rsqrt(jnp.where(dg_ref[1] > 0, dg_ref[1], 1.0))
        ns_ref[...] = ns
        nd_ref[...] = nd
        hs = h_ref[...] * ns
        for t in range(NET):
            out_ref[t] = hs * et_ref[t]

    return pl.pallas_call(
        body,
        grid=(GRID,),
        in_specs=[pl.BlockSpec((RB, D), lambda i: (i, 0)),
                  pl.BlockSpec((NCORES, RB, D), lambda i: (0, i, 0)),
                  pl.BlockSpec(memory_space=pltpu.SMEM)],
        out_specs=[pl.BlockSpec((NET, RB, D), lambda i: (0, i, 0)),
                   pl.BlockSpec((RB, D), lambda i: (i, 0)),
                   pl.BlockSpec((RB, D), lambda i: (i, 0))],
        out_shape=[jax.ShapeDtypeStruct((NET, NPAD, D), jnp.float32),
                   jax.ShapeDtypeStruct((NPAD, D), jnp.float32),
                   jax.ShapeDtypeStruct((NPAD, D), jnp.float32)],
    )(h0, deg, et)


def _tc_table_l1(agg, nd, ns, et):
    def body(a_ref, nd_ref, ns_ref, et_ref, out_ref):
        hs = (a_ref[0] + a_ref[1]) * nd_ref[...] * ns_ref[...]
        for t in range(NET):
            out_ref[t] = hs * et_ref[t]

    return pl.pallas_call(
        body,
        grid=(GRID,),
        in_specs=[pl.BlockSpec((NCORES, RB, D), lambda i: (0, i, 0)),
                  pl.BlockSpec((RB, D), lambda i: (i, 0)),
                  pl.BlockSpec((RB, D), lambda i: (i, 0)),
                  pl.BlockSpec(memory_space=pltpu.SMEM)],
        out_specs=pl.BlockSpec((NET, RB, D), lambda i: (0, i, 0)),
        out_shape=jax.ShapeDtypeStruct((NET, NPAD, D), jnp.float32),
    )(agg, nd, ns, et)


def _tc_table_l2(agg, nd, w1, b1, ns, et):
    def body(a_ref, nd_ref, w_ref, b_ref, ns_ref, et_ref, out_ref):
        hin = (a_ref[0] + a_ref[1]) * nd_ref[...]
        h = jnp.dot(hin, w_ref[...], preferred_element_type=jnp.float32) + b_ref[...]
        hs = jnp.maximum(h, 0.0) * ns_ref[...]
        for t in range(NET):
            out_ref[t] = hs * et_ref[t]

    return pl.pallas_call(
        body,
        grid=(GRID,),
        in_specs=[pl.BlockSpec((NCORES, RB, D), lambda i: (0, i, 0)),
                  pl.BlockSpec((RB, D), lambda i: (i, 0)),
                  pl.BlockSpec((D, D), lambda i: (0, 0)),
                  pl.BlockSpec((1, D), lambda i: (0, 0)),
                  pl.BlockSpec((RB, D), lambda i: (i, 0)),
                  pl.BlockSpec(memory_space=pltpu.SMEM)],
        out_specs=pl.BlockSpec((NET, RB, D), lambda i: (0, i, 0)),
        out_shape=jax.ShapeDtypeStruct((NET, NPAD, D), jnp.float32),
    )(agg, nd, w1, b1.reshape(1, D), ns, et)


def _tc_final(agg, nd, w2p, b2p):
    def body(a_ref, nd_ref, w_ref, b_ref, out_ref):
        hin = (a_ref[0] + a_ref[1]) * nd_ref[...]
        out_ref[...] = jnp.dot(hin, w_ref[...],
                               preferred_element_type=jnp.float32) + b_ref[...]

    return pl.pallas_call(
        body,
        grid=(GRID,),
        in_specs=[pl.BlockSpec((NCORES, RB, D), lambda i: (0, i, 0)),
                  pl.BlockSpec((RB, D), lambda i: (i, 0)),
                  pl.BlockSpec((D, D), lambda i: (0, 0)),
                  pl.BlockSpec((1, D), lambda i: (0, 0))],
        out_specs=pl.BlockSpec((RB, D), lambda i: (i, 0)),
        out_shape=jax.ShapeDtypeStruct((NPAD, D), jnp.float32),
    )(agg, nd, w2p, b2p.reshape(1, D))


def kernel(x0, edge_index, e_feat, W_fc0, b_fc0, et0, et1, et2, W1, b1, W2, b2):
    src = edge_index[0]
    dst = edge_index[1]
    pad = EP - E
    padidx = NPAD + (jnp.arange(pad, dtype=jnp.int32) % 64)
    src2 = jnp.concatenate([src, padidx])
    dst2 = jnp.concatenate([dst, padidx])
    ef2 = jnp.concatenate([e_feat, jnp.zeros((pad,), jnp.int32)])
    x0p = jnp.pad(x0, ((0, NPAD - N), (0, 0)))
    w2p = jnp.pad(W2, ((0, 0), (0, D - NCLS)))
    b2p = jnp.pad(b2, ((0, D - NCLS),))
    ones_in = jnp.ones((CHUNK, D), jnp.float32)
    zeros_in = jnp.zeros((ZROWS, D), jnp.float32)

    deg, gidx2 = _sc_prep(src2, dst2, ef2, ones_in, zeros_in)
    h0 = _tc_h0(x0p, W_fc0.T, b_fc0)
    tbl0, ns, nd = _tc_t0(h0, deg, et0)

    agg0 = _sc_conv(tbl0.reshape(NET * NPAD, D), gidx2, dst2)
    tbl1 = _tc_table_l1(agg0, nd, ns, et1).reshape(NET * NPAD, D)
    agg1 = _sc_conv(tbl1, gidx2, dst2)
    tbl2 = _tc_table_l2(agg1, nd, W1, b1, ns, et2).reshape(NET * NPAD, D)
    agg2 = _sc_conv(tbl2, gidx2, dst2)
    out = _tc_final(agg2, nd, w2p, b2p)
    return out[:N, :NCLS]
